# Initial kernel scaffold; baseline (speedup 1.0000x reference)
import functools

import numpy as np
import jax
import jax.numpy as jnp
from jax import lax
from jax.experimental import pallas as pl
from jax.experimental.pallas import tpu as pltpu
from jax.experimental.pallas import tpu_sc as plsc

L = 24
NC, NS, LN = 2, 16, 16
NW = NC * NS
TPAD = 2304
DUMP = 16
RRAW = TPAD + DUMP


def _build_static():
    widths = []
    for level in range(1, L):
        for _pos in range(L - level):
            widths.append(level)
    widths = np.asarray(widths, np.int64)
    n_seg = len(widths)
    offs = np.concatenate([[0], np.cumsum(widths)[:-1]])

    perm, seg_id, seg_off, mdum = [], [], [], []
    ppos = {}
    for p in range(int(widths.max())):
        for s in np.nonzero(widths > p)[0]:
            ppos[(s, p)] = len(perm)
            perm.append(offs[s] + p)
            seg_id.append(s)
            seg_off.append(offs[s])
            mdum.append(1.0)
        while len(perm) % LN:
            t = len(perm)
            perm.append(0)
            seg_id.append(n_seg + (t % LN))
            seg_off.append(TPAD + (t % LN))
            mdum.append(0.0)
    t2 = len(perm)
    sid = np.asarray(seg_id)
    for c in range(t2 // LN):
        assert len(set(sid[c * LN:(c + 1) * LN].tolist())) == LN

    pi, pj = [], []
    for s in range(n_seg):
        for i in range(int(widths[s])):
            for j in range(i + 1, int(widths[s])):
                pi.append(ppos[(s, i)])
                pj.append(ppos[(s, j)])
    while len(pi) % LN:
        pi.append(0)
        pj.append(0)
    nbin = -(-(n_seg + LN) // LN) * LN
    return dict(
        t2=t2, npair=len(pi), nbin=nbin,
        perm=np.asarray(perm, np.int32), seg_id=np.asarray(seg_id, np.int32),
        seg_off=np.asarray(seg_off, np.int32),
        mdum=np.asarray(mdum, np.float32),
        pi=np.asarray(pi, np.int32), pj=np.asarray(pj, np.int32))


_ST = _build_static()
T2 = _ST["t2"]
NPAIR = _ST["npair"]
NBIN = _ST["nbin"]
NCHUNK = T2 // LN
NPV = NPAIR // LN


def _sc_loss(sc, ru, mk, ix, w, perm, sid, soff, pi, pj, md):
    b = sc.shape[0]
    rows_per = b // NW
    mesh = plsc.VectorSubcoreMesh(core_axis_name="c", subcore_axis_name="s")
    f32 = jnp.float32
    i32 = jnp.int32

    @functools.partial(
        pl.kernel,
        out_type=jax.ShapeDtypeStruct((NW, LN), f32),
        mesh=mesh,
        scratch_types=[
            pltpu.VMEM((w.shape[0],), f32),
            pltpu.VMEM((T2,), i32),
            pltpu.VMEM((T2,), i32),
            pltpu.VMEM((T2,), i32),
            pltpu.VMEM((NPAIR,), i32),
            pltpu.VMEM((NPAIR,), i32),
            pltpu.VMEM((T2,), f32),
            pltpu.VMEM((TPAD,), f32),
            pltpu.VMEM((TPAD,), i32),
            pltpu.VMEM((TPAD,), f32),
            pltpu.VMEM((TPAD,), i32),
            pltpu.VMEM((T2,), f32),
            pltpu.VMEM((T2,), f32),
            pltpu.VMEM((RRAW,), f32),
            pltpu.VMEM((RRAW,), f32),
            pltpu.VMEM((T2,), f32),
            pltpu.VMEM((T2,), f32),
            pltpu.VMEM((T2,), f32),
            pltpu.VMEM((NBIN,), f32),
            pltpu.VMEM((NBIN,), f32),
            pltpu.VMEM((LN,), f32),
        ],
    )
    def k(sc_h, ru_h, mk_h, ix_h, w_h, perm_h, sid_h, soff_h, pi_h, pj_h,
          md_h, out_h, w_v, perm_v, sid_v, soff_v, pi_v, pj_v, md_v,
          sc_v, ru_v, mk_v, ix_v, ea_v, er_v, rr_v, dr_v, r_v, a_v, d_v,
          dena_v, denr_v, acc_v):
        wid = lax.axis_index("s") * NC + lax.axis_index("c")
        pltpu.sync_copy(w_h, w_v)
        pltpu.sync_copy(perm_h, perm_v)
        pltpu.sync_copy(sid_h, sid_v)
        pltpu.sync_copy(soff_h, soff_v)
        pltpu.sync_copy(pi_h, pi_v)
        pltpu.sync_copy(pj_h, pj_v)
        pltpu.sync_copy(md_h, md_v)
        zero = jnp.zeros((LN,), f32)

        def row_body(rl, acc):
            row = wid * rows_per + rl
            pltpu.sync_copy(sc_h.at[row], sc_v)
            pltpu.sync_copy(ru_h.at[row], ru_v)
            pltpu.sync_copy(mk_h.at[row], mk_v)
            pltpu.sync_copy(ix_h.at[row], ix_v)

            def zero_big(i, c):
                rr_v[pl.ds(i * LN, LN)] = zero
                dr_v[pl.ds(i * LN, LN)] = zero
                return c
            lax.fori_loop(0, RRAW // LN, zero_big, 0)

            def zero_bins(i, c):
                dena_v[pl.ds(i * LN, LN)] = zero
                denr_v[pl.ds(i * LN, LN)] = zero
                return c
            lax.fori_loop(0, NBIN // LN, zero_bins, 0)

            def p1(i, c):
                s = pl.ds(i * LN, LN)
                pm = perm_v[s]
                ea = jnp.exp(plsc.load_gather(sc_v, [pm]))
                ea_v[s] = ea
                plsc.addupdate_scatter(dena_v, [sid_v[s]], ea)
                me = plsc.load_gather(mk_v, [pm]) * md_v[s]
                g = plsc.load_gather(w_v, [plsc.load_gather(ru_v, [pm])]) * me
                tgt = soff_v[s] + plsc.load_gather(ix_v, [pm])
                plsc.addupdate_scatter(rr_v, [tgt], g)
                plsc.addupdate_scatter(dr_v, [tgt], me)
                return c
            lax.fori_loop(0, NCHUNK, p1, 0)

            def p2(i, c):
                s = pl.ds(i * LN, LN)
                er = jnp.exp(jnp.abs(plsc.load_gather(rr_v, [perm_v[s]])))
                er_v[s] = er
                plsc.addupdate_scatter(denr_v, [sid_v[s]], er)
                return c
            lax.fori_loop(0, NCHUNK, p2, 0)

            def p3(i, c):
                s = pl.ds(i * LN, LN)
                sb = sid_v[s]
                r_v[s] = er_v[s] / (plsc.load_gather(denr_v, [sb]) + 1e-10)
                a_v[s] = ea_v[s] / plsc.load_gather(dena_v, [sb])
                dd = plsc.load_gather(dr_v, [perm_v[s]])
                d_v[s] = jnp.where(dd > 0.0, 1.0, 0.0).astype(f32)
                return c
            lax.fori_loop(0, NCHUNK, p3, 0)

            def p4(i, a):
                s = pl.ds(i * LN, LN)
                vi = pi_v[s]
                vj = pj_v[s]
                drr = plsc.load_gather(r_v, [vj]) - plsc.load_gather(r_v, [vi])
                daa = plsc.load_gather(a_v, [vj]) - plsc.load_gather(a_v, [vi])
                qr = jnp.exp(jnp.minimum(jnp.maximum(100.0 * drr, -30.0), 30.0))
                qa = jnp.exp(jnp.minimum(jnp.maximum(100.0 * daa, -30.0), 30.0))
                m = plsc.load_gather(d_v, [vi]) * plsc.load_gather(d_v, [vj])
                v = (2.0 * (qr - qa)) / ((qr + 1.0) * (qa + 1.0)) * m
                return a + v * v
            return lax.fori_loop(0, NPV, p4, acc)

        acc = lax.fori_loop(0, rows_per, row_body, zero)
        acc_v[...] = acc
        pltpu.sync_copy(acc_v, out_h.at[wid])

    return k(sc, ru, mk, ix, w, perm, sid, soff, pi, pj, md)


def kernel(sentences, scalars_flat, rules_flat, mask_flat, indices_flat,
           mat_weight):
    b, t = scalars_flat.shape
    pad = TPAD - t
    sc = jnp.pad(scalars_flat.astype(jnp.float32), ((0, 0), (0, pad)))
    ru = jnp.pad(rules_flat.astype(jnp.int32), ((0, 0), (0, pad)))
    mk = jnp.pad(mask_flat.astype(jnp.float32), ((0, 0), (0, pad)))
    ix = jnp.pad(indices_flat.astype(jnp.int32), ((0, 0), (0, pad)))
    w = jnp.pad(mat_weight[:, 0].astype(jnp.float32),
                (0, 2504 - mat_weight.shape[0]))
    out = _sc_loss(sc, ru, mk, ix, w,
                   jnp.asarray(_ST["perm"]), jnp.asarray(_ST["seg_id"]),
                   jnp.asarray(_ST["seg_off"]), jnp.asarray(_ST["pi"]),
                   jnp.asarray(_ST["pj"]), jnp.asarray(_ST["mdum"]))
    return 0.01 * jnp.sum(out)


# speedup vs baseline: 134.5383x; 134.5383x over previous
import functools

import numpy as np
import jax
import jax.numpy as jnp
from jax import lax
from jax.experimental import pallas as pl
from jax.experimental.pallas import tpu as pltpu
from jax.experimental.pallas import tpu_sc as plsc

L = 24
NC, NS, LN = 2, 16, 16
NW = NC * NS
TPAD = 2304
DUMP = 16
RRAW = TPAD + DUMP


def _build_static():
    widths = []
    for level in range(1, L):
        for _pos in range(L - level):
            widths.append(level)
    widths = np.asarray(widths, np.int64)
    n_seg = len(widths)
    offs = np.concatenate([[0], np.cumsum(widths)[:-1]])

    perm, seg_id, seg_off, mdum = [], [], [], []
    ppos = {}
    for p in range(int(widths.max())):
        for s in np.nonzero(widths > p)[0]:
            ppos[(s, p)] = len(perm)
            perm.append(offs[s] + p)
            seg_id.append(s)
            seg_off.append(offs[s])
            mdum.append(1.0)
        while len(perm) % LN:
            t = len(perm)
            perm.append(0)
            seg_id.append(n_seg + (t % LN))
            seg_off.append(TPAD + (t % LN))
            mdum.append(0.0)
    t2 = len(perm)
    sid = np.asarray(seg_id)
    for c in range(t2 // LN):
        assert len(set(sid[c * LN:(c + 1) * LN].tolist())) == LN

    pi, pj = [], []
    for s in range(n_seg):
        for i in range(int(widths[s])):
            for j in range(i + 1, int(widths[s])):
                pi.append(ppos[(s, i)])
                pj.append(ppos[(s, j)])
    while len(pi) % LN:
        pi.append(0)
        pj.append(0)
    nbin = -(-(n_seg + LN) // LN) * LN
    return dict(
        t2=t2, npair=len(pi), nbin=nbin,
        perm=np.asarray(perm, np.int32), seg_id=np.asarray(seg_id, np.int32),
        seg_off=np.asarray(seg_off, np.int32),
        mdum=np.asarray(mdum, np.float32),
        pi=np.asarray(pi, np.int32), pj=np.asarray(pj, np.int32))


_ST = _build_static()
T2 = _ST["t2"]
NPAIR = _ST["npair"]
NBIN = _ST["nbin"]
NCHUNK = T2 // LN
NPV = NPAIR // LN


def _sc_loss(sc, ru, mk, ix, w, perm, sid, soff, pi, pj, md):
    b = sc.shape[0]
    rows_per = b // NW
    mesh = plsc.VectorSubcoreMesh(core_axis_name="c", subcore_axis_name="s")
    f32 = jnp.float32
    i32 = jnp.int32

    @functools.partial(
        pl.kernel,
        out_type=jax.ShapeDtypeStruct((NW, LN), f32),
        mesh=mesh,
        compiler_params=pltpu.CompilerParams(needs_layout_passes=False),
        scratch_types=[
            pltpu.VMEM((w.shape[0],), f32),
            pltpu.VMEM((T2,), i32),
            pltpu.VMEM((T2,), i32),
            pltpu.VMEM((T2,), i32),
            pltpu.VMEM((NPAIR,), i32),
            pltpu.VMEM((NPAIR,), i32),
            pltpu.VMEM((T2,), f32),
            pltpu.VMEM((TPAD,), f32),
            pltpu.VMEM((TPAD,), i32),
            pltpu.VMEM((TPAD,), f32),
            pltpu.VMEM((TPAD,), i32),
            pltpu.VMEM((T2,), f32),
            pltpu.VMEM((T2,), f32),
            pltpu.VMEM((RRAW,), f32),
            pltpu.VMEM((RRAW,), f32),
            pltpu.VMEM((T2,), f32),
            pltpu.VMEM((T2,), f32),
            pltpu.VMEM((T2,), f32),
            pltpu.VMEM((NBIN,), f32),
            pltpu.VMEM((NBIN,), f32),
            pltpu.VMEM((LN,), f32),
        ],
    )
    def k(sc_h, ru_h, mk_h, ix_h, w_h, perm_h, sid_h, soff_h, pi_h, pj_h,
          md_h, out_h, w_v, perm_v, sid_v, soff_v, pi_v, pj_v, md_v,
          sc_v, ru_v, mk_v, ix_v, ea_v, er_v, rr_v, dr_v, r_v, a_v, d_v,
          dena_v, denr_v, acc_v):
        wid = lax.axis_index("s") * NC + lax.axis_index("c")
        pltpu.sync_copy(w_h, w_v)
        pltpu.sync_copy(perm_h, perm_v)
        pltpu.sync_copy(sid_h, sid_v)
        pltpu.sync_copy(soff_h, soff_v)
        pltpu.sync_copy(pi_h, pi_v)
        pltpu.sync_copy(pj_h, pj_v)
        pltpu.sync_copy(md_h, md_v)
        zero = jnp.zeros((LN,), f32)

        def row_body(rl, acc):
            row = wid * rows_per + rl
            pltpu.sync_copy(sc_h.at[row], sc_v)
            pltpu.sync_copy(ru_h.at[row], ru_v)
            pltpu.sync_copy(mk_h.at[row], mk_v)
            pltpu.sync_copy(ix_h.at[row], ix_v)

            def zero_big(i, c):
                rr_v[pl.ds(i * LN, LN)] = zero
                dr_v[pl.ds(i * LN, LN)] = zero
                return c
            lax.fori_loop(0, RRAW // LN, zero_big, 0)

            def zero_bins(i, c):
                dena_v[pl.ds(i * LN, LN)] = zero
                denr_v[pl.ds(i * LN, LN)] = zero
                return c
            lax.fori_loop(0, NBIN // LN, zero_bins, 0)

            def p1(i, c):
                s = pl.ds(i * LN, LN)
                pm = perm_v[s]
                ea = jnp.exp(plsc.load_gather(sc_v, [pm]))
                ea_v[s] = ea
                plsc.addupdate_scatter(dena_v, [sid_v[s]], ea)
                me = plsc.load_gather(mk_v, [pm]) * md_v[s]
                g = plsc.load_gather(w_v, [plsc.load_gather(ru_v, [pm])]) * me
                tgt = soff_v[s] + plsc.load_gather(ix_v, [pm])
                plsc.addupdate_scatter(rr_v, [tgt], g)
                plsc.addupdate_scatter(dr_v, [tgt], me)
                return c
            lax.fori_loop(0, NCHUNK, p1, 0)

            def p2(i, c):
                s = pl.ds(i * LN, LN)
                er = jnp.exp(jnp.abs(plsc.load_gather(rr_v, [perm_v[s]])))
                er_v[s] = er
                plsc.addupdate_scatter(denr_v, [sid_v[s]], er)
                return c
            lax.fori_loop(0, NCHUNK, p2, 0)

            def p3(i, c):
                s = pl.ds(i * LN, LN)
                sb = sid_v[s]
                r_v[s] = er_v[s] / (plsc.load_gather(denr_v, [sb]) + 1e-10)
                a_v[s] = ea_v[s] / plsc.load_gather(dena_v, [sb])
                dd = plsc.load_gather(dr_v, [perm_v[s]])
                d_v[s] = jnp.where(dd > 0.0, 1.0, 0.0).astype(f32)
                return c
            lax.fori_loop(0, NCHUNK, p3, 0)

            def p4(i, a):
                s = pl.ds(i * LN, LN)
                vi = pi_v[s]
                vj = pj_v[s]
                drr = plsc.load_gather(r_v, [vj]) - plsc.load_gather(r_v, [vi])
                daa = plsc.load_gather(a_v, [vj]) - plsc.load_gather(a_v, [vi])
                qr = jnp.exp(jnp.minimum(jnp.maximum(100.0 * drr, -30.0), 30.0))
                qa = jnp.exp(jnp.minimum(jnp.maximum(100.0 * daa, -30.0), 30.0))
                m = plsc.load_gather(d_v, [vi]) * plsc.load_gather(d_v, [vj])
                v = (2.0 * (qr - qa)) / ((qr + 1.0) * (qa + 1.0)) * m
                return a + v * v
            return lax.fori_loop(0, NPV, p4, acc)

        acc = lax.fori_loop(0, rows_per, row_body, zero)
        acc_v[...] = acc
        pltpu.sync_copy(acc_v, out_h.at[wid])

    return k(sc, ru, mk, ix, w, perm, sid, soff, pi, pj, md)


def kernel(sentences, scalars_flat, rules_flat, mask_flat, indices_flat,
           mat_weight):
    b, t = scalars_flat.shape
    pad = TPAD - t
    sc = jnp.pad(scalars_flat.astype(jnp.float32), ((0, 0), (0, pad)))
    ru = jnp.pad(rules_flat.astype(jnp.int32), ((0, 0), (0, pad)))
    mk = jnp.pad(mask_flat.astype(jnp.float32), ((0, 0), (0, pad)))
    ix = jnp.pad(indices_flat.astype(jnp.int32), ((0, 0), (0, pad)))
    w = jnp.pad(mat_weight[:, 0].astype(jnp.float32),
                (0, 2504 - mat_weight.shape[0]))
    out = _sc_loss(sc, ru, mk, ix, w,
                   jnp.asarray(_ST["perm"]), jnp.asarray(_ST["seg_id"]),
                   jnp.asarray(_ST["seg_off"]), jnp.asarray(_ST["pi"]),
                   jnp.asarray(_ST["pj"]), jnp.asarray(_ST["mdum"]))
    return 0.01 * jnp.sum(out)


# speedup vs baseline: 166.8055x; 1.2398x over previous
import functools

import numpy as np
import jax
import jax.numpy as jnp
from jax import lax
from jax.experimental import pallas as pl
from jax.experimental.pallas import tpu as pltpu
from jax.experimental.pallas import tpu_sc as plsc

L = 24
NC, NS, LN = 2, 16, 16
NW = NC * NS
TPAD = 2304
DUMP = 16
RRAW = TPAD + DUMP


def _build_static():
    widths = []
    for level in range(1, L):
        for _pos in range(L - level):
            widths.append(level)
    widths = np.asarray(widths, np.int64)
    n_seg = len(widths)
    offs = np.concatenate([[0], np.cumsum(widths)[:-1]])

    perm, seg_id, seg_off, mdum = [], [], [], []
    ppos = {}
    for p in range(int(widths.max())):
        for s in np.nonzero(widths > p)[0]:
            ppos[(s, p)] = len(perm)
            perm.append(offs[s] + p)
            seg_id.append(s)
            seg_off.append(offs[s])
            mdum.append(1.0)
        while len(perm) % LN:
            t = len(perm)
            perm.append(0)
            seg_id.append(n_seg + (t % LN))
            seg_off.append(TPAD + (t % LN))
            mdum.append(0.0)
    t2 = len(perm)
    sid = np.asarray(seg_id)
    for c in range(t2 // LN):
        assert len(set(sid[c * LN:(c + 1) * LN].tolist())) == LN

    pi, pj = [], []
    for s in range(n_seg):
        for i in range(int(widths[s])):
            for j in range(i + 1, int(widths[s])):
                pi.append(ppos[(s, i)])
                pj.append(ppos[(s, j)])
    while len(pi) % LN:
        pi.append(0)
        pj.append(0)
    nbin = -(-(n_seg + LN) // LN) * LN
    return dict(
        t2=t2, npair=len(pi), nbin=nbin,
        perm=np.asarray(perm, np.int32), seg_id=np.asarray(seg_id, np.int32),
        seg_off=np.asarray(seg_off, np.int32),
        mdum=np.asarray(mdum, np.float32),
        pi=np.asarray(pi, np.int32), pj=np.asarray(pj, np.int32))


_ST = _build_static()
T2 = _ST["t2"]
NPAIR = _ST["npair"]
NBIN = _ST["nbin"]
NCHUNK = T2 // LN
NPV = NPAIR // LN


def _sc_loss(sc, ru, mk, ix, w, perm, sid, soff, pi, pj, md):
    b = sc.shape[0]
    rows_per = b // NW
    mesh = plsc.VectorSubcoreMesh(core_axis_name="c", subcore_axis_name="s")
    f32 = jnp.float32
    i32 = jnp.int32

    @functools.partial(
        pl.kernel,
        out_type=jax.ShapeDtypeStruct((NW, LN), f32),
        mesh=mesh,
        compiler_params=pltpu.CompilerParams(needs_layout_passes=False),
        scratch_types=[
            pltpu.VMEM((w.shape[0],), f32),
            pltpu.VMEM((T2,), i32),
            pltpu.VMEM((T2,), i32),
            pltpu.VMEM((T2,), i32),
            pltpu.VMEM((NPAIR,), i32),
            pltpu.VMEM((NPAIR,), i32),
            pltpu.VMEM((T2,), f32),
            pltpu.VMEM((TPAD,), f32),
            pltpu.VMEM((TPAD,), i32),
            pltpu.VMEM((TPAD,), f32),
            pltpu.VMEM((TPAD,), i32),
            pltpu.VMEM((T2,), f32),
            pltpu.VMEM((T2,), f32),
            pltpu.VMEM((RRAW,), f32),
            pltpu.VMEM((RRAW,), f32),
            pltpu.VMEM((T2,), f32),
            pltpu.VMEM((T2,), f32),
            pltpu.VMEM((T2,), f32),
            pltpu.VMEM((NBIN,), f32),
            pltpu.VMEM((NBIN,), f32),
            pltpu.VMEM((LN,), f32),
        ],
    )
    def k(sc_h, ru_h, mk_h, ix_h, w_h, perm_h, sid_h, soff_h, pi_h, pj_h,
          md_h, out_h, w_v, perm_v, sid_v, soff_v, pi_v, pj_v, md_v,
          sc_v, ru_v, mk_v, ix_v, ea_v, er_v, rr_v, dr_v, r_v, a_v, d_v,
          dena_v, denr_v, acc_v):
        wid = lax.axis_index("s") * NC + lax.axis_index("c")
        pltpu.sync_copy(w_h, w_v)
        pltpu.sync_copy(perm_h, perm_v)
        pltpu.sync_copy(sid_h, sid_v)
        pltpu.sync_copy(soff_h, soff_v)
        pltpu.sync_copy(pi_h, pi_v)
        pltpu.sync_copy(pj_h, pj_v)
        pltpu.sync_copy(md_h, md_v)
        zero = jnp.zeros((LN,), f32)

        def row_body(rl, acc):
            row = wid * rows_per + rl
            pltpu.sync_copy(sc_h.at[row], sc_v)
            pltpu.sync_copy(ru_h.at[row], ru_v)
            pltpu.sync_copy(mk_h.at[row], mk_v)
            pltpu.sync_copy(ix_h.at[row], ix_v)

            def zero_big(i):
                rr_v[pl.ds(i * LN, LN)] = zero
                dr_v[pl.ds(i * LN, LN)] = zero
            plsc.parallel_loop(0, RRAW // LN, unroll=4)(zero_big)

            def zero_bins(i):
                dena_v[pl.ds(i * LN, LN)] = zero
                denr_v[pl.ds(i * LN, LN)] = zero
            plsc.parallel_loop(0, NBIN // LN, unroll=4)(zero_bins)

            def p1(i):
                s = pl.ds(i * LN, LN)
                pm = perm_v[s]
                ea = jnp.exp(plsc.load_gather(sc_v, [pm]))
                ea_v[s] = ea
                plsc.addupdate_scatter(dena_v, [sid_v[s]], ea)
                me = plsc.load_gather(mk_v, [pm]) * md_v[s]
                g = plsc.load_gather(w_v, [plsc.load_gather(ru_v, [pm])]) * me
                tgt = soff_v[s] + plsc.load_gather(ix_v, [pm])
                plsc.addupdate_scatter(rr_v, [tgt], g)
                plsc.addupdate_scatter(dr_v, [tgt], me)
            plsc.parallel_loop(0, NCHUNK, unroll=4)(p1)

            def p2(i):
                s = pl.ds(i * LN, LN)
                er = jnp.exp(jnp.abs(plsc.load_gather(rr_v, [perm_v[s]])))
                er_v[s] = er
                plsc.addupdate_scatter(denr_v, [sid_v[s]], er)
            plsc.parallel_loop(0, NCHUNK, unroll=4)(p2)

            def p3(i):
                s = pl.ds(i * LN, LN)
                sb = sid_v[s]
                r_v[s] = er_v[s] / (plsc.load_gather(denr_v, [sb]) + 1e-10)
                a_v[s] = ea_v[s] / plsc.load_gather(dena_v, [sb])
                dd = plsc.load_gather(dr_v, [perm_v[s]])
                d_v[s] = jnp.where(dd > 0.0, 1.0, 0.0).astype(f32)
            plsc.parallel_loop(0, NCHUNK, unroll=4)(p3)

            def p4(i, a):
                s = pl.ds(i * LN, LN)
                vi = pi_v[s]
                vj = pj_v[s]
                drr = plsc.load_gather(r_v, [vj]) - plsc.load_gather(r_v, [vi])
                daa = plsc.load_gather(a_v, [vj]) - plsc.load_gather(a_v, [vi])
                qr = jnp.exp(jnp.minimum(jnp.maximum(100.0 * drr, -30.0), 30.0))
                qa = jnp.exp(jnp.minimum(jnp.maximum(100.0 * daa, -30.0), 30.0))
                m = plsc.load_gather(d_v, [vi]) * plsc.load_gather(d_v, [vj])
                v = (2.0 * (qr - qa)) / ((qr + 1.0) * (qa + 1.0)) * m
                return a + v * v
            return plsc.parallel_loop(0, NPV, unroll=8, carry=acc)(p4)

        acc = lax.fori_loop(0, rows_per, row_body, zero)
        acc_v[...] = acc
        pltpu.sync_copy(acc_v, out_h.at[wid])

    return k(sc, ru, mk, ix, w, perm, sid, soff, pi, pj, md)


def kernel(sentences, scalars_flat, rules_flat, mask_flat, indices_flat,
           mat_weight):
    b, t = scalars_flat.shape
    pad = TPAD - t
    sc = jnp.pad(scalars_flat.astype(jnp.float32), ((0, 0), (0, pad)))
    ru = jnp.pad(rules_flat.astype(jnp.int32), ((0, 0), (0, pad)))
    mk = jnp.pad(mask_flat.astype(jnp.float32), ((0, 0), (0, pad)))
    ix = jnp.pad(indices_flat.astype(jnp.int32), ((0, 0), (0, pad)))
    w = jnp.pad(mat_weight[:, 0].astype(jnp.float32),
                (0, 2504 - mat_weight.shape[0]))
    out = _sc_loss(sc, ru, mk, ix, w,
                   jnp.asarray(_ST["perm"]), jnp.asarray(_ST["seg_id"]),
                   jnp.asarray(_ST["seg_off"]), jnp.asarray(_ST["pi"]),
                   jnp.asarray(_ST["pj"]), jnp.asarray(_ST["mdum"]))
    return 0.01 * jnp.sum(out)
